# baseline (speedup 1.0000x reference)
import functools

import jax
import jax.numpy as jnp
from jax import lax
from jax.experimental import pallas as pl

_TOP_K = 8


def _router_block(x_ref, wt_ref, b_ref, gates_ref, idx_ref):
    logits = jnp.dot(
        x_ref[...], wt_ref[...], preferred_element_type=jnp.float32
    )
    lt = logits.T + b_ref[...]
    ne, bt = lt.shape
    iota = lax.broadcasted_iota(jnp.int32, (ne, bt), 0)
    neg_inf = jnp.float32(-jnp.inf)
    cur = lt
    vals = []
    idxs = []
    for _ in range(_TOP_K):
        m = jnp.max(cur, axis=0, keepdims=True)
        hit = cur == m
        i = jnp.min(jnp.where(hit, iota, ne), axis=0, keepdims=True)
        vals.append(m)
        idxs.append(i)
        cur = jnp.where(iota == i, neg_inf, cur)
    topv = jnp.concatenate(vals, axis=0)
    topi = jnp.concatenate(idxs, axis=0)
    e = jnp.exp(topv - topv[:1])
    gates_ref[...] = e / jnp.sum(e, axis=0, keepdims=True)
    idx_ref[...] = topi


@jax.jit
def kernel(x, W, b):
    B, S, D = x.shape
    E = W.shape[0]
    T = B * S
    x2 = x.reshape(T, D)
    bt = 512
    while T % bt:
        bt //= 2
    grid = (T // bt,)
    gates_t, idx_t = pl.pallas_call(
        _router_block,
        grid=grid,
        in_specs=[
            pl.BlockSpec((bt, D), lambda i: (i, 0)),
            pl.BlockSpec((D, E), lambda i: (0, 0)),
            pl.BlockSpec((E, 1), lambda i: (0, 0)),
        ],
        out_specs=[
            pl.BlockSpec((_TOP_K, bt), lambda i: (0, i)),
            pl.BlockSpec((_TOP_K, bt), lambda i: (0, i)),
        ],
        out_shape=[
            jax.ShapeDtypeStruct((_TOP_K, T), jnp.float32),
            jax.ShapeDtypeStruct((_TOP_K, T), jnp.int32),
        ],
    )(x2, W.T, b.reshape(E, 1))
    return (gates_t.T.reshape(B, S, _TOP_K),
            idx_t.T.reshape(B, S, _TOP_K))
